# Initial kernel scaffold; baseline (speedup 1.0000x reference)
import jax
import jax.numpy as jnp
from jax.experimental import pallas as pl


def kernel(x_a, edge_index_a, edge_attr_a, batch_a, enz_a, x_b, edge_index_b, edge_attr_b, batch_b, enz_b, params):
    raise NotImplementedError("write your pallas kernel here")


# speedup vs baseline: 1.0936x; 1.0936x over previous
import functools

import jax
import jax.numpy as jnp
from jax import lax
from jax.experimental import pallas as pl
from jax.experimental.pallas import tpu as pltpu

N_NODES = 10000
N_PAD = 10240
N_EDGES = 160000
E_PAD = 163840
NODE_DIM = 128
EDGE_DIM = 16
EMB = 64
HEADS = 4
ENZ = 15
N_GRAPHS = 64
NDW = 80


def _k1a_body(x_ref, w_ref, b_ref, y_ref, ps_ref, pq_ref):
    y = jnp.dot(x_ref[...], w_ref[...], preferred_element_type=jnp.float32)
    y = y + b_ref[...]
    y_ref[...] = y
    ps_ref[...] = jnp.broadcast_to(jnp.sum(y, axis=0, keepdims=True), (1, 8, EMB))
    pq_ref[...] = jnp.broadcast_to(jnp.sum(y * y, axis=0, keepdims=True), (1, 8, EMB))


def _k1a(x, W0, b0):
    grid = 10
    blk = N_NODES // grid
    return pl.pallas_call(
        _k1a_body,
        grid=(grid,),
        in_specs=[
            pl.BlockSpec((blk, NODE_DIM), lambda i: (i, 0)),
            pl.BlockSpec((NODE_DIM, EMB), lambda i: (0, 0)),
            pl.BlockSpec((1, EMB), lambda i: (0, 0)),
        ],
        out_specs=[
            pl.BlockSpec((blk, EMB), lambda i: (i, 0)),
            pl.BlockSpec((1, 8, EMB), lambda i: (i, 0, 0)),
            pl.BlockSpec((1, 8, EMB), lambda i: (i, 0, 0)),
        ],
        out_shape=[
            jax.ShapeDtypeStruct((N_NODES, EMB), jnp.float32),
            jax.ShapeDtypeStruct((grid, 8, EMB), jnp.float32),
            jax.ShapeDtypeStruct((grid, 8, EMB), jnp.float32),
        ],
    )(x, W0, b0.reshape(1, -1))


def _k2a_body(y_ref, m_ref, v_ref, g_ref, be_ref, wl_ref, bl_ref, wr_ref,
              br_ref, we_ref, att_ref, smcnt_ref, xl_ref, xr_ref, nd_ref):
    y = y_ref[...]
    h0 = (y - m_ref[...]) * lax.rsqrt(v_ref[...] + 1e-5) * g_ref[...] + be_ref[...]
    h0 = jnp.maximum(h0, 0.0)
    sm = smcnt_ref[0, :, 0:EDGE_DIM] + smcnt_ref[1, :, 0:EDGE_DIM]
    cnt = smcnt_ref[0, :, EDGE_DIM:EDGE_DIM + 1] + smcnt_ref[1, :, EDGE_DIM:EDGE_DIM + 1]
    la = sm / jnp.maximum(cnt, 1.0)
    for h in range(HEADS):
        c0, c1 = h * EMB, (h + 1) * EMB
        xl_h = jnp.dot(h0, wl_ref[:, c0:c1], preferred_element_type=jnp.float32) + bl_ref[:, c0:c1]
        xr_h = jnp.dot(h0, wr_ref[:, c0:c1], preferred_element_type=jnp.float32) + br_ref[:, c0:c1]
        epl = jnp.dot(la, we_ref[:, c0:c1], preferred_element_type=jnp.float32)
        vv = xl_h + xr_h + epl
        vv = jnp.maximum(vv, 0.2 * vv)
        alpha = jnp.sum(vv * att_ref[h:h + 1, :], axis=1, keepdims=True)
        ex = jnp.exp(alpha)
        xl_ref[h] = xl_h
        xr_ref[h] = xr_h
        nd_ref[h] = jnp.concatenate(
            [ex * xl_h, ex, jnp.zeros((xl_h.shape[0], NDW - EMB - 1), jnp.float32)], axis=1)


def _k2a(y_pad, m, v, g0, be0, Wl1, bl1, Wr1, br1, We1, att1, smcnt):
    grid = 16
    blk = N_PAD // grid
    return pl.pallas_call(
        _k2a_body,
        grid=(grid,),
        in_specs=[
            pl.BlockSpec((blk, EMB), lambda i: (i, 0)),
            pl.BlockSpec((1, EMB), lambda i: (0, 0)),
            pl.BlockSpec((1, EMB), lambda i: (0, 0)),
            pl.BlockSpec((1, EMB), lambda i: (0, 0)),
            pl.BlockSpec((1, EMB), lambda i: (0, 0)),
            pl.BlockSpec((EMB, HEADS * EMB), lambda i: (0, 0)),
            pl.BlockSpec((1, HEADS * EMB), lambda i: (0, 0)),
            pl.BlockSpec((EMB, HEADS * EMB), lambda i: (0, 0)),
            pl.BlockSpec((1, HEADS * EMB), lambda i: (0, 0)),
            pl.BlockSpec((EDGE_DIM, HEADS * EMB), lambda i: (0, 0)),
            pl.BlockSpec((HEADS, EMB), lambda i: (0, 0)),
            pl.BlockSpec((2, blk, 2 * EDGE_DIM), lambda i: (0, i, 0)),
        ],
        out_specs=[
            pl.BlockSpec((HEADS, blk, EMB), lambda i: (0, i, 0)),
            pl.BlockSpec((HEADS, blk, EMB), lambda i: (0, i, 0)),
            pl.BlockSpec((HEADS, blk, NDW), lambda i: (0, i, 0)),
        ],
        out_shape=[
            jax.ShapeDtypeStruct((HEADS, N_PAD, EMB), jnp.float32),
            jax.ShapeDtypeStruct((HEADS, N_PAD, EMB), jnp.float32),
            jax.ShapeDtypeStruct((HEADS, N_PAD, NDW), jnp.float32),
        ],
    )(y_pad, m.reshape(1, -1), v.reshape(1, -1), g0.reshape(1, -1),
      be0.reshape(1, -1), Wl1, bl1.reshape(1, -1), Wr1, br1.reshape(1, -1),
      We1, att1, smcnt)


def _k2b_body(ea_ref, we_ref, ep_ref):
    heads = ep_ref.shape[0]
    for h in range(heads):
        ep_ref[h] = jnp.dot(ea_ref[...], we_ref[:, h * EMB:(h + 1) * EMB],
                            preferred_element_type=jnp.float32)


def _k2b(ea_pad, We, heads):
    grid = 80
    blk = E_PAD // grid
    return pl.pallas_call(
        _k2b_body,
        grid=(grid,),
        in_specs=[
            pl.BlockSpec((blk, EDGE_DIM), lambda i: (i, 0)),
            pl.BlockSpec((EDGE_DIM, heads * EMB), lambda i: (0, 0)),
        ],
        out_specs=pl.BlockSpec((heads, blk, EMB), lambda i: (0, i, 0)),
        out_shape=jax.ShapeDtypeStruct((heads, E_PAD, EMB), jnp.float32),
    )(ea_pad, We)


def _k3_body(nd_ref, bias_ref, wl_ref, bl_ref, wr_ref, br_ref, att_ref,
             xl_ref, xr_ref, nd2_ref):
    hs = []
    for h in range(HEADS):
        num = nd_ref[h, :, 0:EMB]
        den = nd_ref[h, :, EMB:EMB + 1]
        hs.append(num / jnp.maximum(den, 1e-16))
    h1 = jnp.concatenate(hs, axis=1) + bias_ref[...]
    h1 = jnp.where(h1 > 0, h1, jnp.exp(jnp.minimum(h1, 0.0)) - 1.0)
    xl2 = jnp.dot(h1, wl_ref[...], preferred_element_type=jnp.float32) + bl_ref[...]
    xr2 = jnp.dot(h1, wr_ref[...], preferred_element_type=jnp.float32) + br_ref[...]
    vv = xl2 + xr2
    vv = jnp.maximum(vv, 0.2 * vv)
    alpha = jnp.sum(vv * att_ref[...], axis=1, keepdims=True)
    ex = jnp.exp(alpha)
    xl_ref[...] = xl2
    xr_ref[...] = xr2
    nd2_ref[...] = jnp.concatenate(
        [ex * xl2, ex, jnp.zeros((xl2.shape[0], NDW - EMB - 1), jnp.float32)], axis=1)


def _k3(nd_final, bias1, Wl2, bl2, Wr2, br2, att2):
    grid = 16
    blk = N_PAD // grid
    return pl.pallas_call(
        _k3_body,
        grid=(grid,),
        in_specs=[
            pl.BlockSpec((HEADS, blk, NDW), lambda i: (0, i, 0)),
            pl.BlockSpec((1, HEADS * EMB), lambda i: (0, 0)),
            pl.BlockSpec((HEADS * EMB, EMB), lambda i: (0, 0)),
            pl.BlockSpec((1, EMB), lambda i: (0, 0)),
            pl.BlockSpec((HEADS * EMB, EMB), lambda i: (0, 0)),
            pl.BlockSpec((1, EMB), lambda i: (0, 0)),
            pl.BlockSpec((1, EMB), lambda i: (0, 0)),
        ],
        out_specs=[
            pl.BlockSpec((blk, EMB), lambda i: (i, 0)),
            pl.BlockSpec((blk, EMB), lambda i: (i, 0)),
            pl.BlockSpec((blk, NDW), lambda i: (i, 0)),
        ],
        out_shape=[
            jax.ShapeDtypeStruct((N_PAD, EMB), jnp.float32),
            jax.ShapeDtypeStruct((N_PAD, EMB), jnp.float32),
            jax.ShapeDtypeStruct((N_PAD, NDW), jnp.float32),
        ],
    )(nd_final, bias1.reshape(1, -1), Wl2, bl2.reshape(1, -1), Wr2,
      br2.reshape(1, -1), att2)


def _k4_body(nd_ref, bias_ref, batch_ref, ps_ref, pm_ref, pc_ref):
    i = pl.program_id(0)
    nd = nd_ref[0] + nd_ref[1]
    h2 = nd[:, 0:EMB] / jnp.maximum(nd[:, EMB:EMB + 1], 1e-16) + bias_ref[...]
    h2 = jnp.where(h2 > 0, h2, jnp.exp(jnp.minimum(h2, 0.0)) - 1.0)
    b = batch_ref[0, 0, :]
    gids = lax.broadcasted_iota(jnp.int32, (1, N_GRAPHS), 1)
    oh = (b[:, None] == gids).astype(jnp.float32)
    ps = lax.dot_general(oh, h2, (((0,), (0,)), ((), ())),
                         preferred_element_type=jnp.float32)
    pc = jnp.sum(oh, axis=0).reshape(N_GRAPHS, 1)
    mxs = []
    for g in range(N_GRAPHS):
        mg = jnp.max(jnp.where(b[:, None] == g, h2, -jnp.inf), axis=0)
        mxs.append(mg.reshape(1, EMB))
    mx = jnp.concatenate(mxs, axis=0)

    @pl.when(i == 0)
    def _():
        ps_ref[...] = jnp.zeros_like(ps_ref)
        pm_ref[...] = jnp.full_like(pm_ref, -jnp.inf)
        pc_ref[...] = jnp.zeros_like(pc_ref)

    ps_ref[...] += ps
    pm_ref[...] = jnp.maximum(pm_ref[...], mx)
    pc_ref[...] += pc


def _k4(nd2_parts, bias2, batch3):
    grid = 10
    blk = N_NODES // grid
    return pl.pallas_call(
        _k4_body,
        grid=(grid,),
        in_specs=[
            pl.BlockSpec((2, blk, NDW), lambda i: (0, i, 0)),
            pl.BlockSpec((1, EMB), lambda i: (0, 0)),
            pl.BlockSpec((1, 1, blk), lambda i: (i, 0, 0)),
        ],
        out_specs=[
            pl.BlockSpec((N_GRAPHS, EMB), lambda i: (0, 0)),
            pl.BlockSpec((N_GRAPHS, EMB), lambda i: (0, 0)),
            pl.BlockSpec((N_GRAPHS, 1), lambda i: (0, 0)),
        ],
        out_shape=[
            jax.ShapeDtypeStruct((N_GRAPHS, EMB), jnp.float32),
            jax.ShapeDtypeStruct((N_GRAPHS, EMB), jnp.float32),
            jax.ShapeDtypeStruct((N_GRAPHS, 1), jnp.float32),
        ],
    )(nd2_parts, bias2.reshape(1, -1), batch3)


def _k5_body(psa_ref, pma_ref, pca_ref, psb_ref, pmb_ref, pcb_ref,
             ea_ref, eb_ref, wh1_ref, bh1_ref, gh_ref, beh_ref, wh2_ref,
             bh2_ref, wh3_ref, bh3_ref, wp1_ref, bp1_ref, wp2_ref, bp2_ref,
             wg1_ref, bg1_ref, wg2_ref, bg2_ref, out_ref):
    def pool(ps, pm, pc):
        mean = ps / jnp.maximum(pc, 1.0)
        mx = jnp.where(pc > 0, pm, 0.0)
        return jnp.concatenate([mean, mx], axis=1)

    va = pool(psa_ref[...], pma_ref[...], pca_ref[...])
    vb = pool(psb_ref[...], pmb_ref[...], pcb_ref[...])
    gat_c = jnp.concatenate([va + vb, jnp.abs(va - vb), va * vb], axis=1)
    ena, enb = ea_ref[...], eb_ref[...]
    enz_c = jnp.concatenate([ena + enb, jnp.abs(ena - enb), ena * enb], axis=1)
    y = jnp.dot(gat_c, wh1_ref[...], preferred_element_type=jnp.float32) + bh1_ref[...]
    m = jnp.mean(y, axis=0, keepdims=True)
    v = jnp.mean((y - m) * (y - m), axis=0, keepdims=True)
    y = (y - m) * lax.rsqrt(v + 1e-5) * gh_ref[...] + beh_ref[...]
    h = jnp.maximum(y, 0.0)
    h = jnp.maximum(jnp.dot(h, wh2_ref[...], preferred_element_type=jnp.float32) + bh2_ref[...], 0.0)
    gat_logits = jnp.dot(h, wh3_ref[...], preferred_element_type=jnp.float32) + bh3_ref[...]
    hp = jnp.maximum(jnp.dot(enz_c, wp1_ref[...], preferred_element_type=jnp.float32) + bp1_ref[...], 0.0)
    prior_logits = jnp.dot(hp, wp2_ref[...], preferred_element_type=jnp.float32) + bp2_ref[...]
    gate_in = jnp.concatenate([gat_c, enz_c], axis=1)
    hg = jnp.maximum(jnp.dot(gate_in, wg1_ref[...], preferred_element_type=jnp.float32) + bg1_ref[...], 0.0)
    al = jax.nn.sigmoid(jnp.dot(hg, wg2_ref[...], preferred_element_type=jnp.float32) + bg2_ref[...])
    out_ref[...] = al * gat_logits + (1.0 - al) * prior_logits


def _k5(pa, pb, enz_a, enz_b, p):
    args = (pa[0], pa[1], pa[2], pb[0], pb[1], pb[2], enz_a, enz_b,
            p['Wh1'], p['bh1'].reshape(1, -1), p['gh'].reshape(1, -1),
            p['beh'].reshape(1, -1), p['Wh2'], p['bh2'].reshape(1, -1),
            p['Wh3'], p['bh3'].reshape(1, -1), p['Wp1'], p['bp1'].reshape(1, -1),
            p['Wp2'], p['bp2'].reshape(1, -1), p['Wg1'], p['bg1'].reshape(1, -1),
            p['Wg2'], p['bg2'].reshape(1, -1))
    return pl.pallas_call(
        _k5_body,
        out_shape=jax.ShapeDtypeStruct((N_GRAPHS, 1), jnp.float32),
    )(*args)


def _sc_fillmean(dst_p, ea32_p):
    acc = jax.ops.segment_sum(ea32_p, dst_p, num_segments=N_PAD)
    return jnp.stack([acc, jnp.zeros_like(acc)], axis=0)


def _sc_edges_l1(src_p, dst_p, xl1, xr1, ep1, nd_init):
    xlg = xl1[:, src_p, :]
    xrg = xr1[:, dst_p, :]
    vv = xlg + xrg + ep1
    vv = jnp.maximum(vv, 0.2 * vv)
    return nd_init, vv


def _edges_accum(src_p, dst_p, xl_t, xr_t, ep_t, att, nd_init):
    xlg = jnp.take(xl_t, src_p, axis=1)
    xrg = jnp.take(xr_t, dst_p, axis=1)
    vv = xlg + xrg + ep_t
    vv = jnp.maximum(vv, 0.2 * vv)
    alpha = jnp.sum(vv * att[:, None, :], axis=-1)
    ex = jnp.exp(alpha)
    num = jax.vmap(lambda e, x, d: jax.ops.segment_sum(e[:, None] * x, d, num_segments=N_PAD),
                   in_axes=(0, 0, None))(ex, xlg, dst_p)
    den = jax.vmap(lambda e, d: jax.ops.segment_sum(e, d, num_segments=N_PAD),
                   in_axes=(0, None))(ex, dst_p)
    pad = jnp.zeros((num.shape[0], N_PAD, NDW - EMB - 1), jnp.float32)
    nd_edges = jnp.concatenate([num, den[:, :, None], pad], axis=-1)
    return nd_init + nd_edges


def _arm(x, edge_index, edge_attr, batch, p):
    src_p = jnp.concatenate([edge_index[0], jnp.zeros((E_PAD - N_EDGES,), jnp.int32)])
    dst_p = jnp.concatenate([edge_index[1], jnp.full((E_PAD - N_EDGES,), N_NODES, jnp.int32)])
    ea_pad = jnp.concatenate([edge_attr, jnp.zeros((E_PAD - N_EDGES, EDGE_DIM), jnp.float32)], axis=0)
    ones_col = jnp.concatenate([jnp.ones((N_EDGES, 1), jnp.float32),
                                jnp.zeros((E_PAD - N_EDGES, 1), jnp.float32)], axis=0)
    ea32 = jnp.concatenate([ea_pad, ones_col,
                            jnp.zeros((E_PAD, EDGE_DIM - 1), jnp.float32)], axis=1)

    y, ps, pq = _k1a(x, p['W0'], p['b0'])
    m = jnp.sum(ps[:, 0, :], axis=0) / N_NODES
    v = jnp.sum(pq[:, 0, :], axis=0) / N_NODES - m * m
    y_pad = jnp.concatenate([y, jnp.zeros((N_PAD - N_NODES, EMB), jnp.float32)], axis=0)

    smcnt = _sc_fillmean(dst_p, ea32)

    xl1, xr1, nd1 = _k2a(y_pad, m, v, p['g0'], p['be0'], p['Wl1'], p['bl1'],
                         p['Wr1'], p['br1'], p['We1'], p['att1'], smcnt)
    ep1 = _k2b(ea_pad, p['We1'], HEADS)

    nd1_final = _edges_accum(src_p, dst_p, xl1, xr1, ep1, p['att1'], nd1)

    xl2, xr2, nd2 = _k3(nd1_final, p['bias1'], p['Wl2'], p['bl2'], p['Wr2'],
                        p['br2'], p['att2'])
    ep2 = _k2b(ea_pad, p['We2'], 1)

    nd2_final = _edges_accum(src_p, dst_p, xl2[None], xr2[None], ep2,
                             p['att2'], nd2[None])[0]
    nd2_parts = jnp.stack([nd2_final, jnp.zeros_like(nd2_final)], axis=0)

    batch3 = batch.reshape(10, 1, N_NODES // 10)
    return _k4(nd2_parts[:, :N_NODES, :], p['bias2'], batch3)


def kernel(x_a, edge_index_a, edge_attr_a, batch_a, enz_a,
           x_b, edge_index_b, edge_attr_b, batch_b, enz_b, params):
    pa = _arm(x_a, edge_index_a, edge_attr_a, batch_a, params)
    pb = _arm(x_b, edge_index_b, edge_attr_b, batch_b, params)
    return _k5(pa, pb, enz_a, enz_b, params)


# speedup vs baseline: 7.0908x; 6.4842x over previous
import functools

import jax
import jax.numpy as jnp
from jax import lax
from jax.experimental import pallas as pl
from jax.experimental.pallas import tpu as pltpu

N_NODES = 10000
N_PAD = 10240
N_EDGES = 160000
E_PAD = 163840
NODE_DIM = 128
EDGE_DIM = 16
EMB = 64
HEADS = 4
ENZ = 15
N_GRAPHS = 64
NDW = 80


def _k1a_body(x_ref, w_ref, b_ref, y_ref, ps_ref, pq_ref):
    y = jnp.dot(x_ref[...], w_ref[...], preferred_element_type=jnp.float32)
    y = y + b_ref[...]
    y_ref[...] = y
    ps_ref[...] = jnp.broadcast_to(jnp.sum(y, axis=0, keepdims=True), (1, 8, EMB))
    pq_ref[...] = jnp.broadcast_to(jnp.sum(y * y, axis=0, keepdims=True), (1, 8, EMB))


def _k1a(x, W0, b0):
    grid = 10
    blk = N_NODES // grid
    return pl.pallas_call(
        _k1a_body,
        grid=(grid,),
        in_specs=[
            pl.BlockSpec((blk, NODE_DIM), lambda i: (i, 0)),
            pl.BlockSpec((NODE_DIM, EMB), lambda i: (0, 0)),
            pl.BlockSpec((1, EMB), lambda i: (0, 0)),
        ],
        out_specs=[
            pl.BlockSpec((blk, EMB), lambda i: (i, 0)),
            pl.BlockSpec((1, 8, EMB), lambda i: (i, 0, 0)),
            pl.BlockSpec((1, 8, EMB), lambda i: (i, 0, 0)),
        ],
        out_shape=[
            jax.ShapeDtypeStruct((N_NODES, EMB), jnp.float32),
            jax.ShapeDtypeStruct((grid, 8, EMB), jnp.float32),
            jax.ShapeDtypeStruct((grid, 8, EMB), jnp.float32),
        ],
    )(x, W0, b0.reshape(1, -1))


def _k2a_body(y_ref, m_ref, v_ref, g_ref, be_ref, wl_ref, bl_ref, wr_ref,
              br_ref, we_ref, att_ref, smcnt_ref, xl_ref, xr_ref, nd_ref):
    y = y_ref[...]
    h0 = (y - m_ref[...]) * lax.rsqrt(v_ref[...] + 1e-5) * g_ref[...] + be_ref[...]
    h0 = jnp.maximum(h0, 0.0)
    sm = smcnt_ref[0, :, 0:EDGE_DIM] + smcnt_ref[1, :, 0:EDGE_DIM]
    cnt = smcnt_ref[0, :, EDGE_DIM:EDGE_DIM + 1] + smcnt_ref[1, :, EDGE_DIM:EDGE_DIM + 1]
    la = sm / jnp.maximum(cnt, 1.0)
    for h in range(HEADS):
        c0, c1 = h * EMB, (h + 1) * EMB
        xl_h = jnp.dot(h0, wl_ref[:, c0:c1], preferred_element_type=jnp.float32) + bl_ref[:, c0:c1]
        xr_h = jnp.dot(h0, wr_ref[:, c0:c1], preferred_element_type=jnp.float32) + br_ref[:, c0:c1]
        epl = jnp.dot(la, we_ref[:, c0:c1], preferred_element_type=jnp.float32)
        vv = xl_h + xr_h + epl
        vv = jnp.maximum(vv, 0.2 * vv)
        alpha = jnp.sum(vv * att_ref[h:h + 1, :], axis=1, keepdims=True)
        ex = jnp.exp(alpha)
        xl_ref[h] = xl_h
        xr_ref[h] = xr_h
        nd_ref[h] = jnp.concatenate(
            [ex * xl_h, ex, jnp.zeros((xl_h.shape[0], NDW - EMB - 1), jnp.float32)], axis=1)


def _k2a(y_pad, m, v, g0, be0, Wl1, bl1, Wr1, br1, We1, att1, smcnt):
    grid = 16
    blk = N_PAD // grid
    return pl.pallas_call(
        _k2a_body,
        grid=(grid,),
        in_specs=[
            pl.BlockSpec((blk, EMB), lambda i: (i, 0)),
            pl.BlockSpec((1, EMB), lambda i: (0, 0)),
            pl.BlockSpec((1, EMB), lambda i: (0, 0)),
            pl.BlockSpec((1, EMB), lambda i: (0, 0)),
            pl.BlockSpec((1, EMB), lambda i: (0, 0)),
            pl.BlockSpec((EMB, HEADS * EMB), lambda i: (0, 0)),
            pl.BlockSpec((1, HEADS * EMB), lambda i: (0, 0)),
            pl.BlockSpec((EMB, HEADS * EMB), lambda i: (0, 0)),
            pl.BlockSpec((1, HEADS * EMB), lambda i: (0, 0)),
            pl.BlockSpec((EDGE_DIM, HEADS * EMB), lambda i: (0, 0)),
            pl.BlockSpec((HEADS, EMB), lambda i: (0, 0)),
            pl.BlockSpec((2, blk, 2 * EDGE_DIM), lambda i: (0, i, 0)),
        ],
        out_specs=[
            pl.BlockSpec((HEADS, blk, EMB), lambda i: (0, i, 0)),
            pl.BlockSpec((HEADS, blk, EMB), lambda i: (0, i, 0)),
            pl.BlockSpec((HEADS, blk, NDW), lambda i: (0, i, 0)),
        ],
        out_shape=[
            jax.ShapeDtypeStruct((HEADS, N_PAD, EMB), jnp.float32),
            jax.ShapeDtypeStruct((HEADS, N_PAD, EMB), jnp.float32),
            jax.ShapeDtypeStruct((HEADS, N_PAD, NDW), jnp.float32),
        ],
    )(y_pad, m.reshape(1, -1), v.reshape(1, -1), g0.reshape(1, -1),
      be0.reshape(1, -1), Wl1, bl1.reshape(1, -1), Wr1, br1.reshape(1, -1),
      We1, att1, smcnt)


def _k2b_body(ea_ref, we_ref, ep_ref):
    heads = ep_ref.shape[0]
    for h in range(heads):
        ep_ref[h] = jnp.dot(ea_ref[...], we_ref[:, h * EMB:(h + 1) * EMB],
                            preferred_element_type=jnp.float32)


def _k2b(ea_pad, We, heads):
    grid = 80
    blk = E_PAD // grid
    return pl.pallas_call(
        _k2b_body,
        grid=(grid,),
        in_specs=[
            pl.BlockSpec((blk, EDGE_DIM), lambda i: (i, 0)),
            pl.BlockSpec((EDGE_DIM, heads * EMB), lambda i: (0, 0)),
        ],
        out_specs=pl.BlockSpec((heads, blk, EMB), lambda i: (0, i, 0)),
        out_shape=jax.ShapeDtypeStruct((heads, E_PAD, EMB), jnp.float32),
    )(ea_pad, We)


def _k3_body(nd_ref, bias_ref, wl_ref, bl_ref, wr_ref, br_ref, att_ref,
             xl_ref, xr_ref, nd2_ref):
    hs = []
    for h in range(HEADS):
        num = nd_ref[h, :, 0:EMB]
        den = nd_ref[h, :, EMB:EMB + 1]
        hs.append(num / jnp.maximum(den, 1e-16))
    h1 = jnp.concatenate(hs, axis=1) + bias_ref[...]
    h1 = jnp.where(h1 > 0, h1, jnp.exp(jnp.minimum(h1, 0.0)) - 1.0)
    xl2 = jnp.dot(h1, wl_ref[...], preferred_element_type=jnp.float32) + bl_ref[...]
    xr2 = jnp.dot(h1, wr_ref[...], preferred_element_type=jnp.float32) + br_ref[...]
    vv = xl2 + xr2
    vv = jnp.maximum(vv, 0.2 * vv)
    alpha = jnp.sum(vv * att_ref[...], axis=1, keepdims=True)
    ex = jnp.exp(alpha)
    xl_ref[...] = xl2
    xr_ref[...] = xr2
    nd2_ref[...] = jnp.concatenate(
        [ex * xl2, ex, jnp.zeros((xl2.shape[0], NDW - EMB - 1), jnp.float32)], axis=1)


def _k3(nd_final, bias1, Wl2, bl2, Wr2, br2, att2):
    grid = 16
    blk = N_PAD // grid
    return pl.pallas_call(
        _k3_body,
        grid=(grid,),
        in_specs=[
            pl.BlockSpec((HEADS, blk, NDW), lambda i: (0, i, 0)),
            pl.BlockSpec((1, HEADS * EMB), lambda i: (0, 0)),
            pl.BlockSpec((HEADS * EMB, EMB), lambda i: (0, 0)),
            pl.BlockSpec((1, EMB), lambda i: (0, 0)),
            pl.BlockSpec((HEADS * EMB, EMB), lambda i: (0, 0)),
            pl.BlockSpec((1, EMB), lambda i: (0, 0)),
            pl.BlockSpec((1, EMB), lambda i: (0, 0)),
        ],
        out_specs=[
            pl.BlockSpec((blk, EMB), lambda i: (i, 0)),
            pl.BlockSpec((blk, EMB), lambda i: (i, 0)),
            pl.BlockSpec((blk, NDW), lambda i: (i, 0)),
        ],
        out_shape=[
            jax.ShapeDtypeStruct((N_PAD, EMB), jnp.float32),
            jax.ShapeDtypeStruct((N_PAD, EMB), jnp.float32),
            jax.ShapeDtypeStruct((N_PAD, NDW), jnp.float32),
        ],
    )(nd_final, bias1.reshape(1, -1), Wl2, bl2.reshape(1, -1), Wr2,
      br2.reshape(1, -1), att2)


def _k4_body(nd_ref, bias_ref, batch_ref, ps_ref, pm_ref, pc_ref):
    i = pl.program_id(0)
    nd = nd_ref[0] + nd_ref[1]
    h2 = nd[:, 0:EMB] / jnp.maximum(nd[:, EMB:EMB + 1], 1e-16) + bias_ref[...]
    h2 = jnp.where(h2 > 0, h2, jnp.exp(jnp.minimum(h2, 0.0)) - 1.0)
    b = batch_ref[0, 0, :]
    gids = lax.broadcasted_iota(jnp.int32, (1, N_GRAPHS), 1)
    oh = (b[:, None] == gids).astype(jnp.float32)
    ps = lax.dot_general(oh, h2, (((0,), (0,)), ((), ())),
                         preferred_element_type=jnp.float32)
    pc = jnp.sum(oh, axis=0).reshape(N_GRAPHS, 1)
    mxs = []
    for g in range(N_GRAPHS):
        mg = jnp.max(jnp.where(b[:, None] == g, h2, -jnp.inf), axis=0)
        mxs.append(mg.reshape(1, EMB))
    mx = jnp.concatenate(mxs, axis=0)

    @pl.when(i == 0)
    def _():
        ps_ref[...] = jnp.zeros_like(ps_ref)
        pm_ref[...] = jnp.full_like(pm_ref, -jnp.inf)
        pc_ref[...] = jnp.zeros_like(pc_ref)

    ps_ref[...] += ps
    pm_ref[...] = jnp.maximum(pm_ref[...], mx)
    pc_ref[...] += pc


def _k4(nd2_parts, bias2, batch3):
    grid = 10
    blk = N_NODES // grid
    return pl.pallas_call(
        _k4_body,
        grid=(grid,),
        in_specs=[
            pl.BlockSpec((2, blk, NDW), lambda i: (0, i, 0)),
            pl.BlockSpec((1, EMB), lambda i: (0, 0)),
            pl.BlockSpec((1, 1, blk), lambda i: (i, 0, 0)),
        ],
        out_specs=[
            pl.BlockSpec((N_GRAPHS, EMB), lambda i: (0, 0)),
            pl.BlockSpec((N_GRAPHS, EMB), lambda i: (0, 0)),
            pl.BlockSpec((N_GRAPHS, 1), lambda i: (0, 0)),
        ],
        out_shape=[
            jax.ShapeDtypeStruct((N_GRAPHS, EMB), jnp.float32),
            jax.ShapeDtypeStruct((N_GRAPHS, EMB), jnp.float32),
            jax.ShapeDtypeStruct((N_GRAPHS, 1), jnp.float32),
        ],
    )(nd2_parts, bias2.reshape(1, -1), batch3)


def _k5_body(psa_ref, pma_ref, pca_ref, psb_ref, pmb_ref, pcb_ref,
             ea_ref, eb_ref, wh1_ref, bh1_ref, gh_ref, beh_ref, wh2_ref,
             bh2_ref, wh3_ref, bh3_ref, wp1_ref, bp1_ref, wp2_ref, bp2_ref,
             wg1_ref, bg1_ref, wg2_ref, bg2_ref, out_ref):
    def pool(ps, pm, pc):
        mean = ps / jnp.maximum(pc, 1.0)
        mx = jnp.where(pc > 0, pm, 0.0)
        return jnp.concatenate([mean, mx], axis=1)

    va = pool(psa_ref[...], pma_ref[...], pca_ref[...])
    vb = pool(psb_ref[...], pmb_ref[...], pcb_ref[...])
    gat_c = jnp.concatenate([va + vb, jnp.abs(va - vb), va * vb], axis=1)
    ena, enb = ea_ref[...], eb_ref[...]
    enz_c = jnp.concatenate([ena + enb, jnp.abs(ena - enb), ena * enb], axis=1)
    y = jnp.dot(gat_c, wh1_ref[...], preferred_element_type=jnp.float32) + bh1_ref[...]
    m = jnp.mean(y, axis=0, keepdims=True)
    v = jnp.mean((y - m) * (y - m), axis=0, keepdims=True)
    y = (y - m) * lax.rsqrt(v + 1e-5) * gh_ref[...] + beh_ref[...]
    h = jnp.maximum(y, 0.0)
    h = jnp.maximum(jnp.dot(h, wh2_ref[...], preferred_element_type=jnp.float32) + bh2_ref[...], 0.0)
    gat_logits = jnp.dot(h, wh3_ref[...], preferred_element_type=jnp.float32) + bh3_ref[...]
    hp = jnp.maximum(jnp.dot(enz_c, wp1_ref[...], preferred_element_type=jnp.float32) + bp1_ref[...], 0.0)
    prior_logits = jnp.dot(hp, wp2_ref[...], preferred_element_type=jnp.float32) + bp2_ref[...]
    gate_in = jnp.concatenate([gat_c, enz_c], axis=1)
    hg = jnp.maximum(jnp.dot(gate_in, wg1_ref[...], preferred_element_type=jnp.float32) + bg1_ref[...], 0.0)
    al = jax.nn.sigmoid(jnp.dot(hg, wg2_ref[...], preferred_element_type=jnp.float32) + bg2_ref[...])
    out_ref[...] = al * gat_logits + (1.0 - al) * prior_logits


def _k5(pa, pb, enz_a, enz_b, p):
    args = (pa[0], pa[1], pa[2], pb[0], pb[1], pb[2], enz_a, enz_b,
            p['Wh1'], p['bh1'].reshape(1, -1), p['gh'].reshape(1, -1),
            p['beh'].reshape(1, -1), p['Wh2'], p['bh2'].reshape(1, -1),
            p['Wh3'], p['bh3'].reshape(1, -1), p['Wp1'], p['bp1'].reshape(1, -1),
            p['Wp2'], p['bp2'].reshape(1, -1), p['Wg1'], p['bg1'].reshape(1, -1),
            p['Wg2'], p['bg2'].reshape(1, -1))
    return pl.pallas_call(
        _k5_body,
        out_shape=jax.ShapeDtypeStruct((N_GRAPHS, 1), jnp.float32),
    )(*args)


from jax.experimental.pallas import tpu_sc as plsc

NC, NS, L = 2, 16, 16
CH = 128
_SC_MESH = dict(core_axis_name="c", subcore_axis_name="s",
                num_cores=NC, num_subcores=NS)
ROWS_T = N_PAD // NS


def _sca_body(dst_ref, ea32_ref, zero_ref, out_ref, dst_v, rows_v, acc):
    c = lax.axis_index("c")
    s = lax.axis_index("s")
    sl = pl.ds(s * ROWS_T, ROWS_T)
    pltpu.sync_copy(zero_ref.at[sl], acc.at[sl])
    plsc.subcore_barrier()
    base0 = (c * NS + s) * (E_PAD // (NC * NS))

    def chunk(j, carry):
        base = base0 + j * CH
        pltpu.sync_copy(dst_ref.at[pl.ds(base, CH)], dst_v)
        pltpu.sync_copy(ea32_ref.at[pl.ds(base, CH)], rows_v)
        pltpu.sync_copy(rows_v, acc.at[dst_v], add=True)
        return carry

    lax.fori_loop(0, E_PAD // (NC * NS) // CH, chunk, 0)
    plsc.subcore_barrier()
    pltpu.sync_copy(acc.at[sl], out_ref.at[c, sl])


def _sc_fillmean(dst_p, ea32_p):
    zero = jnp.zeros((N_PAD, 2 * EDGE_DIM), jnp.float32)
    f = pl.kernel(
        _sca_body,
        out_type=jax.ShapeDtypeStruct((NC, N_PAD, 2 * EDGE_DIM), jnp.float32),
        mesh=plsc.VectorSubcoreMesh(**_SC_MESH),
        compiler_params=pltpu.CompilerParams(use_tc_tiling_on_sc=False),
        scratch_types=[
            pltpu.VMEM((CH,), jnp.int32),
            pltpu.VMEM((CH, 2 * EDGE_DIM), jnp.float32),
            pltpu.VMEM_SHARED((N_PAD, 2 * EDGE_DIM), jnp.float32),
        ],
    )
    return f(dst_p, ea32_p, zero)


def _edge_compute(xlg_v, xrg_v, ep_v, attrow_v, msg_v, mask0, mbuf_v):
    def edge(i, carry):
        m = jnp.zeros((L,), jnp.float32)
        for cb in range(EMB // L):
            slc = pl.ds(cb * L, L)
            v = xlg_v[i, slc] + xrg_v[i, slc] + ep_v[i, slc]
            v = jnp.maximum(v, 0.2 * v)
            m = m + v * attrow_v[slc]
        alpha = m[0]
        for q in range(1, L):
            alpha = alpha + m[q]
        exv = jnp.exp(jnp.full((L,), alpha))
        for cb in range(EMB // L):
            slc = pl.ds(cb * L, L)
            msg_v[i, slc] = exv * xlg_v[i, slc]
        msg_v[i, pl.ds(EMB, L)] = exv * mask0
        return carry

    lax.fori_loop(0, CH, edge, 0)


def _scb_body(src_ref, dst_ref, xlf_ref, xrf_ref, ep_ref, att_ref, ndinit_ref,
              out_ref, src_v, dst_v, adjs_v, adjd_v, xlg_v, xrg_v, ep_v,
              msg_v, attrow_v, mbuf_v, acc0, sem):
    c = lax.axis_index("c")
    s = lax.axis_index("s")
    sl = pl.ds(s * ROWS_T, ROWS_T)
    lane = lax.iota(jnp.int32, L)
    mask0 = jnp.where(lane == 0, 1.0, 0.0).astype(jnp.float32)
    n_chunks = E_PAD // NS // CH
    for hh in range(2):
        h = 2 * c + hh
        pltpu.sync_copy(ndinit_ref.at[h, sl], acc0.at[sl])
        plsc.subcore_barrier()
        pltpu.sync_copy(att_ref.at[h], attrow_v)

        def chunk(j, carry):
            base = s * (E_PAD // NS) + j * CH
            pltpu.sync_copy(src_ref.at[pl.ds(base, CH)], src_v)
            pltpu.sync_copy(dst_ref.at[pl.ds(base, CH)], dst_v)
            off = jnp.full((L,), h * N_PAD, jnp.int32)
            for k in range(CH // L):
                slc = pl.ds(k * L, L)
                adjs_v[slc] = src_v[slc] + off
                adjd_v[slc] = dst_v[slc] + off
            pltpu.async_copy(xlf_ref.at[adjs_v], xlg_v, sem).wait()
            pltpu.async_copy(xrf_ref.at[adjd_v], xrg_v, sem).wait()
            pltpu.sync_copy(ep_ref.at[h, pl.ds(base, CH)], ep_v)
            _edge_compute(xlg_v, xrg_v, ep_v, attrow_v, msg_v, mask0, mbuf_v)
            pltpu.sync_copy(msg_v, acc0.at[dst_v], add=True)
            return carry

        lax.fori_loop(0, n_chunks, chunk, 0)
        plsc.subcore_barrier()
        pltpu.sync_copy(acc0.at[sl], out_ref.at[h, sl])
        plsc.subcore_barrier()


def _scc_body(src_ref, dst_ref, xl_ref, xr_ref, ep_ref, att_ref, ndinit2_ref,
              out_ref, src_v, dst_v, xlg_v, xrg_v, ep_v, msg_v, attrow_v,
              mbuf_v, acc, sem):
    c = lax.axis_index("c")
    s = lax.axis_index("s")
    sl = pl.ds(s * ROWS_T, ROWS_T)
    pltpu.sync_copy(ndinit2_ref.at[c, sl], acc.at[sl])
    plsc.subcore_barrier()
    lane = lax.iota(jnp.int32, L)
    mask0 = jnp.where(lane == 0, 1.0, 0.0).astype(jnp.float32)
    pltpu.sync_copy(att_ref.at[0], attrow_v)
    base0 = (c * NS + s) * (E_PAD // (NC * NS))

    def chunk(j, carry):
        base = base0 + j * CH
        pltpu.sync_copy(src_ref.at[pl.ds(base, CH)], src_v)
        pltpu.sync_copy(dst_ref.at[pl.ds(base, CH)], dst_v)
        pltpu.async_copy(xl_ref.at[src_v], xlg_v, sem).wait()
        pltpu.async_copy(xr_ref.at[dst_v], xrg_v, sem).wait()
        pltpu.sync_copy(ep_ref.at[0, pl.ds(base, CH)], ep_v)
        _edge_compute(xlg_v, xrg_v, ep_v, attrow_v, msg_v, mask0, mbuf_v)
        pltpu.sync_copy(msg_v, acc.at[dst_v], add=True)
        return carry

    lax.fori_loop(0, E_PAD // (NC * NS) // CH, chunk, 0)
    plsc.subcore_barrier()
    pltpu.sync_copy(acc.at[sl], out_ref.at[c, sl])


def _sc_edges1(src_p, dst_p, xl1, xr1, ep1, att1, nd_init):
    f = pl.kernel(
        _scb_body,
        out_type=jax.ShapeDtypeStruct((HEADS, N_PAD, NDW), jnp.float32),
        mesh=plsc.VectorSubcoreMesh(**_SC_MESH),
        compiler_params=pltpu.CompilerParams(use_tc_tiling_on_sc=False),
        scratch_types=[
            pltpu.VMEM((CH,), jnp.int32),
            pltpu.VMEM((CH,), jnp.int32),
            pltpu.VMEM((CH,), jnp.int32),
            pltpu.VMEM((CH,), jnp.int32),
            pltpu.VMEM((CH, EMB), jnp.float32),
            pltpu.VMEM((CH, EMB), jnp.float32),
            pltpu.VMEM((CH, EMB), jnp.float32),
            pltpu.VMEM((CH, NDW), jnp.float32),
            pltpu.VMEM((EMB,), jnp.float32),
            pltpu.VMEM((L,), jnp.float32),
            pltpu.VMEM_SHARED((N_PAD, NDW), jnp.float32),
            pltpu.SemaphoreType.DMA,
        ],
    )
    xlf = xl1.reshape(HEADS * N_PAD, EMB)
    xrf = xr1.reshape(HEADS * N_PAD, EMB)
    return f(src_p, dst_p, xlf, xrf, ep1, att1, nd_init)


def _sc_edges2(src_p, dst_p, xl2, xr2, ep2, att2, nd_init):
    f = pl.kernel(
        _scc_body,
        out_type=jax.ShapeDtypeStruct((NC, N_PAD, NDW), jnp.float32),
        mesh=plsc.VectorSubcoreMesh(**_SC_MESH),
        compiler_params=pltpu.CompilerParams(use_tc_tiling_on_sc=False),
        scratch_types=[
            pltpu.VMEM((CH,), jnp.int32),
            pltpu.VMEM((CH,), jnp.int32),
            pltpu.VMEM((CH, EMB), jnp.float32),
            pltpu.VMEM((CH, EMB), jnp.float32),
            pltpu.VMEM((CH, EMB), jnp.float32),
            pltpu.VMEM((CH, NDW), jnp.float32),
            pltpu.VMEM((EMB,), jnp.float32),
            pltpu.VMEM((L,), jnp.float32),
            pltpu.VMEM_SHARED((N_PAD, NDW), jnp.float32),
            pltpu.SemaphoreType.DMA,
        ],
    )
    ndinit2 = jnp.stack([nd_init, jnp.zeros_like(nd_init)], axis=0)
    return f(src_p, dst_p, xl2, xr2, ep2, att2, ndinit2)


def _arm(x, edge_index, edge_attr, batch, p):
    src_p = jnp.concatenate([edge_index[0], jnp.zeros((E_PAD - N_EDGES,), jnp.int32)])
    dst_p = jnp.concatenate([edge_index[1], jnp.full((E_PAD - N_EDGES,), N_NODES, jnp.int32)])
    ea_pad = jnp.concatenate([edge_attr, jnp.zeros((E_PAD - N_EDGES, EDGE_DIM), jnp.float32)], axis=0)
    ones_col = jnp.concatenate([jnp.ones((N_EDGES, 1), jnp.float32),
                                jnp.zeros((E_PAD - N_EDGES, 1), jnp.float32)], axis=0)
    ea32 = jnp.concatenate([ea_pad, ones_col,
                            jnp.zeros((E_PAD, EDGE_DIM - 1), jnp.float32)], axis=1)

    y, ps, pq = _k1a(x, p['W0'], p['b0'])
    m = jnp.sum(ps[:, 0, :], axis=0) / N_NODES
    v = jnp.sum(pq[:, 0, :], axis=0) / N_NODES - m * m
    y_pad = jnp.concatenate([y, jnp.zeros((N_PAD - N_NODES, EMB), jnp.float32)], axis=0)

    smcnt = _sc_fillmean(dst_p, ea32)

    xl1, xr1, nd1 = _k2a(y_pad, m, v, p['g0'], p['be0'], p['Wl1'], p['bl1'],
                         p['Wr1'], p['br1'], p['We1'], p['att1'], smcnt)
    ep1 = _k2b(ea_pad, p['We1'], HEADS)

    nd1_final = _sc_edges1(src_p, dst_p, xl1, xr1, ep1, p['att1'], nd1)

    xl2, xr2, nd2 = _k3(nd1_final, p['bias1'], p['Wl2'], p['bl2'], p['Wr2'],
                        p['br2'], p['att2'])
    ep2 = _k2b(ea_pad, p['We2'], 1)

    nd2_parts = _sc_edges2(src_p, dst_p, xl2, xr2, ep2, p['att2'], nd2)

    batch3 = batch.reshape(10, 1, N_NODES // 10)
    return _k4(nd2_parts[:, :N_NODES, :], p['bias2'], batch3)


def kernel(x_a, edge_index_a, edge_attr_a, batch_a, enz_a,
           x_b, edge_index_b, edge_attr_b, batch_b, enz_b, params):
    pa = _arm(x_a, edge_index_a, edge_attr_a, batch_a, params)
    pb = _arm(x_b, edge_index_b, edge_attr_b, batch_b, params)
    return _k5(pa, pb, enz_a, enz_b, params)
